# baseline (speedup 1.0000x reference)
import functools

import jax
import jax.numpy as jnp
from jax import lax
from jax.experimental import pallas as pl
from jax.experimental.pallas import tpu as pltpu
from jax.experimental.pallas import tpu_sc as plsc

_BATCH = 4096
_HIST = 200
_DIM = 64
_PAD = 128
_VOCAB = 1000000
_VPAD = 1000064
_B = _BATCH * _HIST
_NW = 32

_NBLK = _VPAD // _PAD
_BLKF = _DIM * _PAD
_BLK_PER_W = -(-_NBLK // _NW)

_B_PER_W = _B // _NW
_CHUNK = 400
_NC = _B_PER_W // _CHUNK
_NBUF = 4
_ROUNDS = _NC // _NBUF
_LAG = 2

_mesh = plsc.VectorSubcoreMesh(core_axis_name="c", subcore_axis_name="s")


def _transpose_body(tt_hbm, tp_hbm, vin0, vin1, vout0, vout1,
                    i0, i1, o0, o1):
    vin = (vin0, vin1)
    vout = (vout0, vout1)
    isem = (i0, i1)
    osem = (o0, o1)
    wid = lax.axis_index("s") * 2 + lax.axis_index("c")
    iota = lax.iota(jnp.int32, 16)
    srow = [16 * k + iota for k in range(4)]

    def blk(j):
        return wid * _BLK_PER_W + j

    def in_start(s, b):
        pltpu.async_copy(tt_hbm.at[:, pl.ds(b * _PAD, _PAD)],
                         vin[s].at[:, pl.ds(0, _PAD)], isem[s])

    def in_wait(s):
        pltpu.make_async_copy(tt_hbm.at[:, pl.ds(0, _PAD)],
                              vin[s].at[:, pl.ds(0, _PAD)], isem[s]).wait()

    def out_start(s, b):
        pltpu.async_copy(vout[s], tp_hbm.at[pl.ds(b * _DIM, _DIM)], osem[s])

    def out_wait(s):
        pltpu.make_async_copy(vout[s], tp_hbm.at[pl.ds(0, _DIM)],
                              osem[s]).wait()

    def transpose(s):
        def row(r, _):
            for h in range(2):
                col = jnp.full((16,), 0, jnp.int32) + (2 * r + h)
                for k in range(4):
                    v = plsc.load_gather(vin[s], [srow[k], col])
                    vout[s][r, pl.ds(64 * h + 16 * k, 16)] = v
            return ()
        lax.fori_loop(0, _DIM, row, (), unroll=8)

    @pl.when(blk(0) < _NBLK)
    def _():
        in_start(0, blk(0))

    def step(i, _):
        for s in range(2):
            j = 2 * i + s
            b = blk(j)
            nb = b + 1

            @pl.when(jnp.logical_and(j + 1 < _BLK_PER_W, nb < _NBLK))
            def _():
                in_start(1 - s, nb)

            @pl.when(jnp.logical_and(j < _BLK_PER_W, b < _NBLK))
            def _():
                in_wait(s)

                @pl.when(j >= 2)
                def _():
                    out_wait(s)

                transpose(s)
                out_start(s, b)
        return ()

    lax.fori_loop(0, (_BLK_PER_W + 1) // 2, step, (), unroll=False)
    for s in range(2):
        @pl.when(blk(s) < _NBLK)
        def _():
            out_wait(s)


_transpose = functools.partial(
    pl.kernel,
    out_type=jax.ShapeDtypeStruct((_VPAD // 2, _PAD), jnp.float32),
    mesh=_mesh,
    scratch_types=[
        pltpu.VMEM((_DIM, _PAD + 1), jnp.float32),
        pltpu.VMEM((_DIM, _PAD + 1), jnp.float32),
        pltpu.VMEM((_DIM, _PAD), jnp.float32),
        pltpu.VMEM((_DIM, _PAD), jnp.float32),
        pltpu.SemaphoreType.DMA,
        pltpu.SemaphoreType.DMA,
        pltpu.SemaphoreType.DMA,
        pltpu.SemaphoreType.DMA,
    ],
    compiler_params=pltpu.CompilerParams(use_tc_tiling_on_sc=True,
                                        needs_layout_passes=False),
)(_transpose_body)


def _gather_body(idx_hbm, table_hbm, out_hbm, idx_v,
                 rows0, rows1, rows2, rows3, g0, g1, g2, g3, w0, w1, w2, w3):
    rows = (rows0, rows1, rows2, rows3)
    gsem = (g0, g1, g2, g3)
    wsem = (w0, w1, w2, w3)
    wid = lax.axis_index("s") * 2 + lax.axis_index("c")
    base = wid * _B_PER_W

    pltpu.sync_copy(idx_hbm.at[pl.ds(base, _B_PER_W)], idx_v)

    def gather_start(s, c):
        pltpu.async_copy(
            table_hbm.at[idx_v.at[pl.ds(c * _CHUNK, _CHUNK)]],
            rows[s], gsem[s])

    def gather_wait(s):
        pltpu.make_async_copy(
            table_hbm.at[idx_v.at[pl.ds(0, _CHUNK)]], rows[s], gsem[s]).wait()

    def wb_start(s, c):
        pltpu.async_copy(
            rows[s],
            out_hbm.at[pl.ds(base + c * _CHUNK, _CHUNK), pl.ds(0, _DIM)],
            wsem[s])

    def wb_wait(s):
        pltpu.make_async_copy(
            rows[s],
            out_hbm.at[pl.ds(0, _CHUNK), pl.ds(0, _DIM)], wsem[s]).wait()

    def round_(i, _):
        for s in range(_NBUF):
            c = _NBUF * i + s
            d = c - _LAG
            t = (s + _NBUF - _LAG) % _NBUF

            @pl.when(c >= _NBUF)
            def _():
                wb_wait(s)

            gather_start(s, c)

            @pl.when(d >= 0)
            def _():
                gather_wait(t)
                wb_start(t, d)
        return ()

    lax.fori_loop(0, _ROUNDS, round_, (), unroll=False)

    for c in range(_NC - _LAG, _NC):
        gather_wait(c % _NBUF)
        wb_start(c % _NBUF, c)
    for s in range(_NBUF):
        wb_wait(s)


_gather = functools.partial(
    pl.kernel,
    out_type=jax.ShapeDtypeStruct((_B, _PAD), jnp.float32),
    mesh=_mesh,
    scratch_types=[
        pltpu.VMEM((_B_PER_W,), jnp.int32),
        pltpu.VMEM((_CHUNK, _DIM), jnp.float32),
        pltpu.VMEM((_CHUNK, _DIM), jnp.float32),
        pltpu.VMEM((_CHUNK, _DIM), jnp.float32),
        pltpu.VMEM((_CHUNK, _DIM), jnp.float32),
        pltpu.SemaphoreType.DMA,
        pltpu.SemaphoreType.DMA,
        pltpu.SemaphoreType.DMA,
        pltpu.SemaphoreType.DMA,
        pltpu.SemaphoreType.DMA,
        pltpu.SemaphoreType.DMA,
        pltpu.SemaphoreType.DMA,
        pltpu.SemaphoreType.DMA,
    ],
    compiler_params=pltpu.CompilerParams(use_tc_tiling_on_sc=False),
)(_gather_body)


@jax.jit
def kernel(inputs, table):
    idx = inputs.reshape(_B)
    tp = _transpose(table.T)
    tab = tp.reshape(_VPAD, _DIM)
    out = _gather(idx, tab)
    out = out.reshape(_BATCH, _HIST, _PAD)[:, :, :_DIM]
    return out


# speedup vs baseline: 2.1168x; 2.1168x over previous
import functools

import jax
import jax.numpy as jnp
from jax import lax
from jax.experimental import pallas as pl
from jax.experimental.pallas import tpu as pltpu
from jax.experimental.pallas import tpu_sc as plsc

_BATCH = 4096
_HIST = 200
_DIM = 64
_PAD = 128
_VOCAB = 1000000
_VPAD = 1000064
_B = _BATCH * _HIST
_NW = 32

_B_PER_W = _B // _NW
_CHUNK = 400
_NC = _B_PER_W // _CHUNK
_NBUF = 4
_ROUNDS = _NC // _NBUF
_LAG = 2

_mesh = plsc.VectorSubcoreMesh(core_axis_name="c", subcore_axis_name="s")


def _gather_body(idx_hbm, table_hbm, out_hbm, idx_v,
                 rows0, rows1, rows2, rows3, g0, g1, g2, g3, w0, w1, w2, w3):
    rows = (rows0, rows1, rows2, rows3)
    gsem = (g0, g1, g2, g3)
    wsem = (w0, w1, w2, w3)
    wid = lax.axis_index("s") * 2 + lax.axis_index("c")
    base = wid * _B_PER_W

    pltpu.sync_copy(idx_hbm.at[pl.ds(base, _B_PER_W)], idx_v)

    def gather_start(s, c):
        pltpu.async_copy(
            table_hbm.at[idx_v.at[pl.ds(c * _CHUNK, _CHUNK)]],
            rows[s], gsem[s])

    def gather_wait(s):
        pltpu.make_async_copy(
            table_hbm.at[idx_v.at[pl.ds(0, _CHUNK)]], rows[s], gsem[s]).wait()

    def wb_start(s, c):
        pltpu.async_copy(
            rows[s],
            out_hbm.at[pl.ds(base + c * _CHUNK, _CHUNK), pl.ds(0, _DIM)],
            wsem[s])

    def wb_wait(s):
        pltpu.make_async_copy(
            rows[s],
            out_hbm.at[pl.ds(0, _CHUNK), pl.ds(0, _DIM)], wsem[s]).wait()

    def round_(i, _):
        for s in range(_NBUF):
            c = _NBUF * i + s
            d = c - _LAG
            t = (s + _NBUF - _LAG) % _NBUF

            @pl.when(c >= _NBUF)
            def _():
                wb_wait(s)

            gather_start(s, c)

            @pl.when(d >= 0)
            def _():
                gather_wait(t)
                wb_start(t, d)
        return ()

    lax.fori_loop(0, _ROUNDS, round_, (), unroll=False)

    for c in range(_NC - _LAG, _NC):
        gather_wait(c % _NBUF)
        wb_start(c % _NBUF, c)
    for s in range(_NBUF):
        wb_wait(s)


_gather = functools.partial(
    pl.kernel,
    out_type=jax.ShapeDtypeStruct((_B, _PAD), jnp.float32),
    mesh=_mesh,
    scratch_types=[
        pltpu.VMEM((_B_PER_W,), jnp.int32),
        pltpu.VMEM((_CHUNK, _DIM), jnp.float32),
        pltpu.VMEM((_CHUNK, _DIM), jnp.float32),
        pltpu.VMEM((_CHUNK, _DIM), jnp.float32),
        pltpu.VMEM((_CHUNK, _DIM), jnp.float32),
        pltpu.SemaphoreType.DMA,
        pltpu.SemaphoreType.DMA,
        pltpu.SemaphoreType.DMA,
        pltpu.SemaphoreType.DMA,
        pltpu.SemaphoreType.DMA,
        pltpu.SemaphoreType.DMA,
        pltpu.SemaphoreType.DMA,
        pltpu.SemaphoreType.DMA,
    ],
    compiler_params=pltpu.CompilerParams(use_tc_tiling_on_sc=False),
)(_gather_body)


@jax.jit
def kernel(inputs, table):
    idx = inputs.reshape(_B) * 2
    tablep = jnp.concatenate(
        [table, jnp.zeros((_VOCAB, _DIM), jnp.float32)], axis=1)
    tab2 = tablep.reshape(_VOCAB * 2, _DIM)
    out = _gather(idx, tab2)
    out = out.reshape(_BATCH, _HIST, _PAD)[:, :, :_DIM]
    return out
